# baseline (device time: 53589 ns/iter reference)
import jax
import jax.numpy as jnp
from jax import lax
from jax.experimental import pallas as pl
from jax.experimental.pallas import tpu as pltpu

N_DEV = 4
N_HOP = N_DEV - 1


def kernel(A, B):
    m, k = A.shape
    _, n = B.shape
    ch = m // N_DEV
    qc = n // 6

    def body(a_ref, b_ref, out_ref, a_bf, b_bf, acc_bf, rs_comm, ag_comm,
             rs_send, rs_recv, ag_send, ag_recv):
        my = lax.axis_index("i")
        left = (my - 1) % N_DEV
        right = (my + 1) % N_DEV
        dsts = (right, left)
        own = ((my + 1) % N_DEV, (my - 1) % N_DEV)

        barrier = pltpu.get_barrier_semaphore()
        for nbr in (left, right):
            pl.semaphore_signal(
                barrier, inc=1,
                device_id=(nbr,), device_id_type=pl.DeviceIdType.MESH,
            )
        pl.semaphore_wait(barrier, 2)

        def rows(c):
            return pl.ds(c * ch, ch)

        def ccols(d, s):
            return pl.ds(d * (n // 2) + s * qc, qc)

        def hcols(d):
            return pl.ds(d * (n // 2), n // 2)

        def cva(c):
            a_bf[rows(c), :] = a_ref[rows(c), :].astype(jnp.bfloat16)

        def cvb(d):
            b_bf[:, hcols(d)] = b_ref[:, hcols(d)].astype(jnp.bfloat16)

        def mmh(c, d):
            acc_bf[rows(c), hcols(d)] = jnp.dot(
                a_bf[rows(c), :], b_bf[:, hcols(d)],
                preferred_element_type=jnp.float32,
            ).astype(jnp.bfloat16)

        def rs_schunk(d, h):
            return (my - h) % N_DEV if d == 0 else (my + h) % N_DEV

        def rs_rchunk(d, h):
            return (my - h - 1) % N_DEV if d == 0 else (my + h + 1) % N_DEV

        def ag_rchunk(d, h):
            return (my - h) % N_DEV if d == 0 else (my + h) % N_DEV

        def rs_rdma(d, s, h):
            return pltpu.make_async_remote_copy(
                src_ref=acc_bf.at[rows(rs_schunk(d, h)), ccols(d, s)],
                dst_ref=rs_comm.at[d, s, h],
                send_sem=rs_send.at[d, s, h],
                recv_sem=rs_recv.at[d, s, h],
                device_id=(dsts[d],),
                device_id_type=pl.DeviceIdType.MESH,
            )

        def ag_rdma(d, s, h):
            src = (
                acc_bf.at[rows(own[d]), ccols(d, s)]
                if h == 0
                else ag_comm.at[d, s, h - 1]
            )
            return pltpu.make_async_remote_copy(
                src_ref=src,
                dst_ref=ag_comm.at[d, s, h],
                send_sem=ag_send.at[d, s, h],
                recv_sem=ag_recv.at[d, s, h],
                device_id=(dsts[d],),
                device_id_type=pl.DeviceIdType.MESH,
            )

        cva(my)
        cvb(0)
        mmh(my, 0)
        for s in (0, 1, 2):
            rs_rdma(0, s, 0).start()
        cvb(1)
        mmh(my, 1)
        for s in (0, 1, 2):
            rs_rdma(1, s, 0).start()
        cva((my - 1) % N_DEV)
        mmh((my - 1) % N_DEV, 0)
        cva((my + 1) % N_DEV)
        mmh((my + 1) % N_DEV, 1)
        for h in range(N_HOP - 1):
            for s in (0, 1, 2):
                for d in (0, 1):
                    r = rs_rdma(d, s, h)
                    r.wait_recv()
                    rc = rs_rchunk(d, h)
                    acc_bf[rows(rc), ccols(d, s)] = (
                        acc_bf[rows(rc), ccols(d, s)] + rs_comm[d, s, h]
                    )
                    rs_rdma(d, s, h + 1).start()
            if h == 0:
                cva((my + 2) % N_DEV)
                mmh((my + 2) % N_DEV, 0)
                mmh((my + 2) % N_DEV, 1)
            else:
                mmh((my + 1) % N_DEV, 0)
                mmh((my - 1) % N_DEV, 1)

        for s in (0, 1, 2):
            for d in (0, 1):
                r = rs_rdma(d, s, N_HOP - 1)
                r.wait_recv()
                q = ccols(d, s)
                acc_bf[rows(own[d]), q] = jnp.maximum(
                    acc_bf[rows(own[d]), q] + rs_comm[d, s, N_HOP - 1], 0.0
                )
                ag_rdma(d, s, 0).start()
                out_ref[rows(own[d]), q] = acc_bf[rows(own[d]), q].astype(
                    jnp.float32
                )

        for h in range(N_HOP):
            for s in (0, 1, 2):
                for d in (0, 1):
                    r = ag_rdma(d, s, h)
                    r.wait_recv()
                    if h + 1 < N_HOP:
                        ag_rdma(d, s, h + 1).start()
                    rc = ag_rchunk(d, h)
                    out_ref[rows(rc), ccols(d, s)] = ag_comm[d, s, h].astype(
                        jnp.float32
                    )
                    r.wait_send()

        for h in range(N_HOP):
            for s in (0, 1, 2):
                for d in (0, 1):
                    rs_rdma(d, s, h).wait_send()

    return pl.pallas_call(
        body,
        out_shape=jax.ShapeDtypeStruct((m, n), jnp.float32),
        in_specs=[
            pl.BlockSpec(memory_space=pltpu.VMEM),
            pl.BlockSpec(memory_space=pltpu.VMEM),
        ],
        out_specs=pl.BlockSpec(memory_space=pltpu.VMEM),
        scratch_shapes=[
            pltpu.VMEM((m, k), jnp.bfloat16),
            pltpu.VMEM((k, n), jnp.bfloat16),
            pltpu.VMEM((m, n), jnp.bfloat16),
            pltpu.VMEM((2, 3, N_HOP, ch, qc), jnp.bfloat16),
            pltpu.VMEM((2, 3, N_HOP, ch, qc), jnp.bfloat16),
            pltpu.SemaphoreType.DMA((2, 3, N_HOP)),
            pltpu.SemaphoreType.DMA((2, 3, N_HOP)),
            pltpu.SemaphoreType.DMA((2, 3, N_HOP)),
            pltpu.SemaphoreType.DMA((2, 3, N_HOP)),
        ],
        compiler_params=pltpu.CompilerParams(collective_id=0),
    )(A, B)
